# baseline (speedup 1.0000x reference)
import jax
import jax.numpy as jnp
from jax.experimental import pallas as pl
from jax.experimental.pallas import tpu as pltpu
from jax.experimental.pallas import tpu_sc as plsc

_W = 128


def _sc_gather(table, idx_flat):
    n = idx_flat.shape[0]
    d = table.shape[1]
    mesh = plsc.VectorSubcoreMesh(core_axis_name="core", subcore_axis_name="subcore")
    idx2 = idx_flat.reshape(1, n)

    @jax.jit
    def run(table, idx2):
        @pl.kernel(
            out_type=jax.ShapeDtypeStruct((n, d), table.dtype),
            mesh=mesh,
            compiler_params=pltpu.CompilerParams(use_tc_tiling_on_sc=False),
        )
        def k(table_hbm, idx_hbm, out_hbm):
            def body(i_vmem, o_vmem):
                pltpu.sync_copy(table_hbm.at[i_vmem.at[0]], o_vmem)

            pltpu.emit_pipeline(
                body,
                grid=(n // _W,),
                in_specs=[pl.BlockSpec((1, _W), index_map=lambda i: (0, i))],
                out_specs=[pl.BlockSpec((_W, d), index_map=lambda i: (i, 0))],
                core_axis_name=("core", "subcore"),
                dimension_semantics=(pltpu.PARALLEL,),
            )(idx_hbm, out_hbm)

        return k(table, idx2)

    return run(table, idx2)


def kernel(x, table):
    b, s = x.shape
    out = _sc_gather(table, x.reshape(b * s))
    return out.reshape(b, s, table.shape[1])


# speedup vs baseline: 1.0754x; 1.0754x over previous
import jax
import jax.numpy as jnp
from jax.experimental import pallas as pl
from jax.experimental.pallas import tpu as pltpu
from jax.experimental.pallas import tpu_sc as plsc

_W = 128
_K = 4


def _sc_gather(table, idx_flat):
    n = idx_flat.shape[0]
    d = table.shape[1]
    mesh = plsc.VectorSubcoreMesh(core_axis_name="core", subcore_axis_name="subcore")
    idx2 = idx_flat.reshape(n // _W, _W)

    @jax.jit
    def run(table, idx2):
        @pl.kernel(
            out_type=jax.ShapeDtypeStruct((n, d), table.dtype),
            mesh=mesh,
            compiler_params=pltpu.CompilerParams(use_tc_tiling_on_sc=False),
        )
        def k(table_hbm, idx_hbm, out_hbm):
            def body(i_vmem, o_vmem, sem):
                copies = [
                    pltpu.make_async_copy(
                        table_hbm.at[i_vmem.at[j]],
                        o_vmem.at[pl.ds(j * _W, _W)],
                        sem,
                    )
                    for j in range(_K)
                ]
                for c in copies:
                    c.start()
                for c in copies:
                    c.wait()

            def body_scoped(i_vmem, o_vmem):
                pl.run_scoped(
                    lambda sem: body(i_vmem, o_vmem, sem),
                    pltpu.SemaphoreType.DMA,
                )

            pltpu.emit_pipeline(
                body_scoped,
                grid=(n // (_K * _W),),
                in_specs=[pl.BlockSpec((_K, _W), index_map=lambda i: (i, 0))],
                out_specs=[pl.BlockSpec((_K * _W, d), index_map=lambda i: (i, 0))],
                core_axis_name=("core", "subcore"),
                dimension_semantics=(pltpu.PARALLEL,),
            )(idx_hbm, out_hbm)

        return k(table, idx2)

    return run(table, idx2)


def kernel(x, table):
    b, s = x.shape
    out = _sc_gather(table, x.reshape(b * s))
    return out.reshape(b, s, table.shape[1])


# speedup vs baseline: 1.3113x; 1.2193x over previous
import jax
import jax.numpy as jnp
from jax.experimental import pallas as pl
from jax.experimental.pallas import tpu as pltpu
from jax.experimental.pallas import tpu_sc as plsc

_W = 128
_K = 2


def _sc_gather(table128, idx2):
    n = idx2.shape[0] * idx2.shape[1]
    mesh = plsc.VectorSubcoreMesh(core_axis_name="core", subcore_axis_name="subcore")

    @pl.kernel(
        out_type=jax.ShapeDtypeStruct((n, 128), table128.dtype),
        mesh=mesh,
    )
    def k(table_hbm, idx_hbm, out_hbm):
        def body(i_vmem, o_vmem, sem):
            copies = [
                pltpu.make_async_copy(
                    table_hbm.at[i_vmem.at[j]],
                    o_vmem.at[pl.ds(j * _W, _W)],
                    sem,
                )
                for j in range(_K)
            ]
            for c in copies:
                c.start()
            for c in copies:
                c.wait()

        def body_scoped(i_vmem, o_vmem):
            pl.run_scoped(
                lambda sem: body(i_vmem, o_vmem, sem),
                pltpu.SemaphoreType.DMA,
            )

        pltpu.emit_pipeline(
            body_scoped,
            grid=(n // (_K * _W),),
            in_specs=[pl.BlockSpec((_K, _W), index_map=lambda i: (i, 0))],
            out_specs=[pl.BlockSpec((_K * _W, 128), index_map=lambda i: (i, 0))],
            core_axis_name=("core", "subcore"),
            dimension_semantics=(pltpu.PARALLEL,),
        )(idx_hbm, out_hbm)

    return k(table128, idx2)


def kernel(x, table):
    b, s = x.shape
    d = table.shape[1]
    table128 = jnp.pad(table, ((0, 0), (0, 128 - d)))
    idx2 = x.reshape((b * s) // _W, _W)
    out = _sc_gather(table128, idx2)
    return out[:, :d].reshape(b, s, d)


# speedup vs baseline: 1.3915x; 1.0612x over previous
import jax
import jax.numpy as jnp
from jax import lax
from jax.experimental import pallas as pl
from jax.experimental.pallas import tpu as pltpu
from jax.experimental.pallas import tpu_sc as plsc

_W = 128
_D = 64


def _sc_gather(table, idx2):
    n_win = idx2.shape[0]
    n = n_win * _W
    info = plsc.get_sparse_core_info()
    nw = info.num_cores * info.num_subcores
    wpw = n_win // nw
    mesh = plsc.VectorSubcoreMesh(core_axis_name="c", subcore_axis_name="s")

    @pl.kernel(
        out_type=jax.ShapeDtypeStruct((n, 2 * _D), jnp.float32),
        mesh=mesh,
        compiler_params=pltpu.CompilerParams(use_tc_tiling_on_sc=False),
        scratch_types=[
            pltpu.VMEM((wpw, _W), jnp.int32),
            pltpu.VMEM((_W, _D), jnp.float32),
            pltpu.VMEM((_W, _D), jnp.float32),
            pltpu.SemaphoreType.DMA,
            pltpu.SemaphoreType.DMA,
            pltpu.SemaphoreType.DMA,
            pltpu.SemaphoreType.DMA,
        ],
    )
    def k(table_hbm, idx_hbm, out_hbm, idxv, g0, g1, s0, s1, t0, t1):
        gs = (g0, g1)
        ss = (s0, s1)
        ts = (t0, t1)
        wid = lax.axis_index("s") * info.num_cores + lax.axis_index("c")
        base = wid * wpw
        pltpu.sync_copy(idx_hbm.at[pl.ds(base, wpw)], idxv)

        def gather(w, slot):
            return pltpu.make_async_copy(
                table_hbm.at[idxv.at[w]], gs[slot], ss[slot])

        def writeback(w, slot):
            return pltpu.make_async_copy(
                gs[slot],
                out_hbm.at[pl.ds((base + w) * _W, _W), pl.ds(0, _D)],
                ts[slot])

        gather(0, 0).start()
        gather(1, 1).start()

        @pl.loop(2, wpw, step=2)
        def _(wo):
            for s in range(2):
                w = wo + s
                gather(w - 2, s).wait()
                writeback(w - 2, s).start()
                writeback(w - 2, s).wait()
                gather(w, s).start()

        for s in range(2):
            w = wpw - 2 + s
            gather(w, s).wait()
            writeback(w, s).start()
            writeback(w, s).wait()

    return k(table, idx2)


def kernel(x, table):
    b, s = x.shape
    d = table.shape[1]
    idx2 = x.reshape((b * s) // _W, _W)
    out = _sc_gather(table, idx2)
    return out[:, :d].reshape(b, s, d)
